# baseline (speedup 1.0000x reference)
import math

import jax
import jax.numpy as jnp
from jax.experimental import pallas as pl
from jax.experimental.pallas import tpu as pltpu

_C_LO = 128
_NUM_1O = 128
_TM = 1024
_VMEM_LIMIT = 32 * 1024 * 1024


def _cdiv(a, b):
    return -(-a // b)


def _readout_body(x1_ref, x2_ref, w_ref, o_ref):
    acc = jnp.dot(x1_ref[...], w_ref[0:128], preferred_element_type=jnp.float32)
    acc += jnp.dot(x2_ref[...], w_ref[128:384], preferred_element_type=jnp.float32)
    o_ref[...] = acc[:, :3]


def kernel(x_flat, weight_1o):
    m, k = x_flat.shape
    num_1o = weight_1o.shape[0]
    assert k == _C_LO + 3 * _NUM_1O and num_1o == _NUM_1O

    wc = weight_1o.astype(jnp.float32) / math.sqrt(float(num_1o))
    w = (wc[:, None, None] * jnp.eye(3, dtype=jnp.float32)[None]).reshape(3 * num_1o, 3)
    w = jnp.pad(w, ((0, 0), (0, 5)))

    tm = min(_TM, m)
    grid = (_cdiv(m, tm),)
    return pl.pallas_call(
        _readout_body,
        out_shape=jax.ShapeDtypeStruct((m, 3), jnp.float32),
        grid_spec=pltpu.PrefetchScalarGridSpec(
            num_scalar_prefetch=0,
            grid=grid,
            in_specs=[
                pl.BlockSpec((tm, 128), lambda i: (i, 1)),
                pl.BlockSpec((tm, 256), lambda i: (i, 1)),
                pl.BlockSpec((3 * num_1o, 8), lambda i: (0, 0)),
            ],
            out_specs=pl.BlockSpec((tm, 3), lambda i: (i, 0)),
        ),
        compiler_params=pltpu.CompilerParams(
            dimension_semantics=("parallel",),
            vmem_limit_bytes=_VMEM_LIMIT),
    )(x_flat, x_flat, w)


# speedup vs baseline: 1.3827x; 1.3827x over previous
import math

import jax
import jax.numpy as jnp
from jax.experimental import pallas as pl
from jax.experimental.pallas import tpu as pltpu

_C_LO = 128
_NUM_1O = 128
_TM = 4096
_VMEM_LIMIT = 32 * 1024 * 1024


def _cdiv(a, b):
    return -(-a // b)


def _readout_body(x1_ref, x2_ref, w_ref, o_ref):
    acc = jnp.dot(x1_ref[...], w_ref[0:128], preferred_element_type=jnp.float32)
    acc += jnp.dot(x2_ref[...], w_ref[128:384], preferred_element_type=jnp.float32)
    o_ref[...] = acc[:, :3]


def kernel(x_flat, weight_1o):
    m, k = x_flat.shape
    num_1o = weight_1o.shape[0]
    assert k == _C_LO + 3 * _NUM_1O and num_1o == _NUM_1O

    wc = weight_1o.astype(jnp.float32) / math.sqrt(float(num_1o))
    w = (wc[:, None, None] * jnp.eye(3, dtype=jnp.float32)[None]).reshape(3 * num_1o, 3)
    w = jnp.pad(w, ((0, 0), (0, 5)))

    tm = min(_TM, m)
    grid = (_cdiv(m, tm),)
    return pl.pallas_call(
        _readout_body,
        out_shape=jax.ShapeDtypeStruct((m, 3), jnp.float32),
        grid_spec=pltpu.PrefetchScalarGridSpec(
            num_scalar_prefetch=0,
            grid=grid,
            in_specs=[
                pl.BlockSpec((tm, 128), lambda i: (i, 1)),
                pl.BlockSpec((tm, 256), lambda i: (i, 1)),
                pl.BlockSpec((3 * num_1o, 8), lambda i: (0, 0)),
            ],
            out_specs=pl.BlockSpec((tm, 3), lambda i: (i, 0)),
        ),
        compiler_params=pltpu.CompilerParams(
            dimension_semantics=("parallel",),
            vmem_limit_bytes=_VMEM_LIMIT),
    )(x_flat, x_flat, w)
